# baseline (speedup 1.0000x reference)
import functools

import jax
import jax.numpy as jnp
from jax import lax
from jax.experimental import pallas as pl
from jax.experimental.pallas import tpu as pltpu
from jax.experimental.pallas import tpu_sc as plsc

VOCAB = 100000
EMB = 64
HID = 64
B = 4096
S = 200

NC = 2
NS = 16
NW = NC * NS
TOK = B * S
TPW = TOK // NW
SUB = 128
KSUB = 10
CHUNK = SUB * KSUB
NCHUNK = TPW // CHUNK

VTILE = 2000


def _project_kernel(emb_ref, wt_ref, b_ref, out_ref):
    out_ref[...] = (
        jnp.dot(emb_ref[...], wt_ref[...], preferred_element_type=jnp.float32)
        + b_ref[...]
    ).astype(out_ref.dtype)


@functools.cache
def _sc_gather_fn():
    mesh = plsc.VectorSubcoreMesh(
        core_axis_name="c", subcore_axis_name="s", num_cores=NC, num_subcores=NS
    )

    @functools.partial(
        pl.kernel,
        out_type=jax.ShapeDtypeStruct((TOK, HID), jnp.bfloat16),
        mesh=mesh,
        scratch_types=[
            pltpu.VMEM((2 * KSUB, SUB), jnp.int32),
            pltpu.VMEM((2 * CHUNK, HID), jnp.bfloat16),
            pltpu.SemaphoreType.DMA((2,)),
            pltpu.SemaphoreType.DMA((2,)),
            pltpu.SemaphoreType.DMA((2,)),
        ],
        compiler_params=pltpu.CompilerParams(use_tc_tiling_on_sc=False),
    )
    def _sc_gather(table_hbm, idx_hbm, out_hbm, idx_v, rows_v, isem, gsem, osem):
        wid = lax.axis_index("s") * NC + lax.axis_index("c")
        row_base = wid * (TPW // SUB)

        def fire_idx(i, b):
            pltpu.async_copy(
                idx_hbm.at[pl.ds(row_base + i * KSUB, KSUB)],
                idx_v.at[pl.ds(b * KSUB, KSUB)],
                isem.at[b],
            )

        def wait_idx(b):
            pltpu.make_async_copy(
                idx_hbm.at[pl.ds(row_base, KSUB)],
                idx_v.at[pl.ds(b * KSUB, KSUB)],
                isem.at[b],
            ).wait()

        def fire_gathers(b):
            for j in range(KSUB):
                pltpu.async_copy(
                    table_hbm.at[idx_v.at[b * KSUB + j]],
                    rows_v.at[pl.ds((b * KSUB + j) * SUB, SUB)],
                    gsem.at[b],
                )

        def wait_gathers(b):
            pltpu.make_async_copy(
                out_hbm.at[pl.ds(0, CHUNK)],
                rows_v.at[pl.ds(b * CHUNK, CHUNK)],
                gsem.at[b],
            ).wait()

        def fire_scatter(i, b):
            pltpu.async_copy(
                rows_v.at[pl.ds(b * CHUNK, CHUNK)],
                out_hbm.at[pl.ds((row_base + i * KSUB) * SUB, CHUNK)],
                osem.at[b],
            )

        def wait_scatter(b):
            pltpu.make_async_copy(
                rows_v.at[pl.ds(b * CHUNK, CHUNK)],
                out_hbm.at[pl.ds(0, CHUNK)],
                osem.at[b],
            ).wait()

        fire_idx(0, 0)
        fire_idx(1, 1)
        wait_idx(0)
        fire_gathers(0)

        @pl.loop(0, NCHUNK)
        def _chunk(i):
            b = lax.rem(i, 2)
            nxt = 1 - b

            @pl.when(i + 1 < NCHUNK)
            def _():
                @pl.when(i >= 1)
                def _():
                    wait_scatter(nxt)

                wait_idx(nxt)
                fire_gathers(nxt)

            wait_gathers(b)

            @pl.when(i + 2 < NCHUNK)
            def _():
                fire_idx(i + 2, b)

            fire_scatter(i, b)

        wait_scatter(0)
        wait_scatter(1)

    return _sc_gather


def _scan_kernel(g_ref, w2_ref, b2_ref, flo_ref, fhi_ref, fcb_ref, out_ref, h_ref):
    t = pl.program_id(0)

    @pl.when(t == 0)
    def _():
        h_ref[...] = jnp.zeros_like(h_ref)

    h_new = jnp.tanh(
        g_ref[...].astype(jnp.float32)
        + jnp.dot(h_ref[...], w2_ref[...], preferred_element_type=jnp.float32)
        + b2_ref[...]
    )
    h_ref[...] = h_new

    @pl.when(t == S - 1)
    def _():
        lo = jnp.sum(h_new * flo_ref[...], axis=1, keepdims=True)
        hi = jnp.sum(h_new * fhi_ref[...], axis=1, keepdims=True)
        logits = jnp.concatenate([lo, hi], axis=1) + fcb_ref[...]
        out_ref[...] = jax.nn.sigmoid(logits)


def _dup_diag(wt):
    z = jnp.zeros_like(wt)
    return jnp.concatenate(
        [jnp.concatenate([wt, z], axis=1), jnp.concatenate([z, wt], axis=1)],
        axis=0,
    )


def kernel(x, emb, W_ih, W_hh, b_ih, b_hh, fc_W, fc_b):
    emb2 = emb.reshape(VOCAB // 2, 2 * EMB)
    w2ih = _dup_diag(W_ih.T)
    b2ih = jnp.concatenate([b_ih, b_ih]).reshape(1, 2 * HID)

    proj2 = pl.pallas_call(
        _project_kernel,
        grid=(VOCAB // 2 // VTILE,),
        in_specs=[
            pl.BlockSpec((VTILE, 2 * EMB), lambda i: (i, 0)),
            pl.BlockSpec((2 * EMB, 2 * HID), lambda i: (0, 0)),
            pl.BlockSpec((1, 2 * HID), lambda i: (0, 0)),
        ],
        out_specs=pl.BlockSpec((VTILE, 2 * HID), lambda i: (i, 0)),
        out_shape=jax.ShapeDtypeStruct((VOCAB // 2, 2 * HID), jnp.bfloat16),
    )(emb2, w2ih, b2ih)

    idx = jnp.transpose(x).reshape(TOK // SUB, SUB)

    g = _sc_gather_fn()(proj2.reshape(VOCAB, HID), idx)
    g2 = g.reshape(TOK // 2, 2 * HID)

    w2hh = _dup_diag(W_hh.T)
    b2hh = jnp.concatenate([b_hh, b_hh]).reshape(1, 2 * HID)
    zf = jnp.zeros((HID,), jnp.float32)
    flo = jnp.concatenate([fc_W[0], zf]).reshape(1, 2 * HID)
    fhi = jnp.concatenate([zf, fc_W[0]]).reshape(1, 2 * HID)

    out = pl.pallas_call(
        _scan_kernel,
        grid=(S,),
        in_specs=[
            pl.BlockSpec((B // 2, 2 * HID), lambda t: (t, 0)),
            pl.BlockSpec((2 * HID, 2 * HID), lambda t: (0, 0)),
            pl.BlockSpec((1, 2 * HID), lambda t: (0, 0)),
            pl.BlockSpec((1, 2 * HID), lambda t: (0, 0)),
            pl.BlockSpec((1, 2 * HID), lambda t: (0, 0)),
            pl.BlockSpec((1, 2), lambda t: (0, 0)),
        ],
        out_specs=pl.BlockSpec((B // 2, 2), lambda t: (0, 0)),
        out_shape=jax.ShapeDtypeStruct((B // 2, 2), jnp.float32),
        scratch_shapes=[pltpu.VMEM((B // 2, 2 * HID), jnp.float32)],
    )(g2, w2hh, b2hh, flo, fhi, jnp.broadcast_to(fc_b.reshape(1, 1), (1, 2)))
    return out.reshape(B)


# speedup vs baseline: 2.0152x; 2.0152x over previous
import functools

import jax
import jax.numpy as jnp
from jax import lax
from jax.experimental import pallas as pl
from jax.experimental.pallas import tpu as pltpu
from jax.experimental.pallas import tpu_sc as plsc

VOCAB = 100000
EMB = 64
HID = 64
B = 4096
S = 200

NC = 2
NS = 16
NW = NC * NS
TOK = B * S
TPW = TOK // NW
SUB = 128
KSUB = 5
CHUNK = SUB * KSUB
NCHUNK = TPW // CHUNK

VTILE = 2000

NSEG = 4
SEG = S // NSEG


def _project_kernel(emb_ref, wt_ref, b_ref, out_ref):
    out_ref[...] = jnp.dot(
        emb_ref[...], wt_ref[...], preferred_element_type=jnp.float32
    ) + b_ref[...]


@functools.cache
def _sc_gather_fn(ntok):
    tpw = ntok // NW
    nchunk = tpw // CHUNK
    mesh = plsc.VectorSubcoreMesh(
        core_axis_name="c", subcore_axis_name="s", num_cores=NC, num_subcores=NS
    )

    @functools.partial(
        pl.kernel,
        out_type=jax.ShapeDtypeStruct((ntok, HID), jnp.float32),
        mesh=mesh,
        scratch_types=[
            pltpu.VMEM((2 * KSUB, SUB), jnp.int32),
            pltpu.VMEM((2 * CHUNK, HID), jnp.float32),
            pltpu.SemaphoreType.DMA((2,)),
            pltpu.SemaphoreType.DMA((2,)),
            pltpu.SemaphoreType.DMA((2,)),
        ],
        compiler_params=pltpu.CompilerParams(use_tc_tiling_on_sc=False),
    )
    def _sc_gather(table_hbm, idx_hbm, out_hbm, idx_v, rows_v, isem, gsem, osem):
        wid = lax.axis_index("s") * NC + lax.axis_index("c")
        row_base = wid * (tpw // SUB)
        NCHUNK = nchunk

        def fire_idx(i, b):
            pltpu.async_copy(
                idx_hbm.at[pl.ds(row_base + i * KSUB, KSUB)],
                idx_v.at[pl.ds(b * KSUB, KSUB)],
                isem.at[b],
            )

        def wait_idx(b):
            pltpu.make_async_copy(
                idx_hbm.at[pl.ds(row_base, KSUB)],
                idx_v.at[pl.ds(b * KSUB, KSUB)],
                isem.at[b],
            ).wait()

        def fire_gathers(b):
            for j in range(KSUB):
                pltpu.async_copy(
                    table_hbm.at[idx_v.at[b * KSUB + j]],
                    rows_v.at[pl.ds((b * KSUB + j) * SUB, SUB)],
                    gsem.at[b],
                )

        def wait_gathers(b):
            pltpu.make_async_copy(
                out_hbm.at[pl.ds(0, CHUNK)],
                rows_v.at[pl.ds(b * CHUNK, CHUNK)],
                gsem.at[b],
            ).wait()

        def fire_scatter(i, b):
            pltpu.async_copy(
                rows_v.at[pl.ds(b * CHUNK, CHUNK)],
                out_hbm.at[pl.ds((row_base + i * KSUB) * SUB, CHUNK)],
                osem.at[b],
            )

        def wait_scatter(b):
            pltpu.make_async_copy(
                rows_v.at[pl.ds(b * CHUNK, CHUNK)],
                out_hbm.at[pl.ds(0, CHUNK)],
                osem.at[b],
            ).wait()

        fire_idx(0, 0)
        fire_idx(1, 1)
        wait_idx(0)
        fire_gathers(0)

        @pl.loop(0, NCHUNK)
        def _chunk(i):
            b = lax.rem(i, 2)
            nxt = 1 - b

            @pl.when(i + 1 < NCHUNK)
            def _():
                @pl.when(i >= 1)
                def _():
                    wait_scatter(nxt)

                wait_idx(nxt)
                fire_gathers(nxt)

            wait_gathers(b)

            @pl.when(i + 2 < NCHUNK)
            def _():
                fire_idx(i + 2, b)

            fire_scatter(i, b)

        wait_scatter(0)
        wait_scatter(1)

    return _sc_gather


def _scan_seg_kernel(
    g_ref, w2_ref, b2_ref, flo_ref, fhi_ref, fcb_ref, hin_ref,
    hout_ref, out_ref, h_ref
):
    t = pl.program_id(0)
    nt = pl.num_programs(0)

    @pl.when(t == 0)
    def _():
        h_ref[...] = hin_ref[...]

    h_new = jnp.tanh(
        g_ref[...]
        + jnp.dot(h_ref[...], w2_ref[...], preferred_element_type=jnp.float32)
        + b2_ref[...]
    )
    h_ref[...] = h_new

    @pl.when(t == nt - 1)
    def _():
        hout_ref[...] = h_new
        lo = jnp.sum(h_new * flo_ref[...], axis=1, keepdims=True)
        hi = jnp.sum(h_new * fhi_ref[...], axis=1, keepdims=True)
        logits = jnp.concatenate([lo, hi], axis=1) + fcb_ref[...]
        out_ref[...] = jax.nn.sigmoid(logits)


def _dup_diag(wt):
    z = jnp.zeros_like(wt)
    return jnp.concatenate(
        [jnp.concatenate([wt, z], axis=1), jnp.concatenate([z, wt], axis=1)],
        axis=0,
    )


def kernel(x, emb, W_ih, W_hh, b_ih, b_hh, fc_W, fc_b):
    emb2 = emb.reshape(VOCAB // 2, 2 * EMB)
    w2ih = _dup_diag(W_ih.T)
    b2ih = jnp.concatenate([b_ih, b_ih]).reshape(1, 2 * HID)

    proj2 = pl.pallas_call(
        _project_kernel,
        grid=(VOCAB // 2 // VTILE,),
        in_specs=[
            pl.BlockSpec((VTILE, 2 * EMB), lambda i: (i, 0)),
            pl.BlockSpec((2 * EMB, 2 * HID), lambda i: (0, 0)),
            pl.BlockSpec((1, 2 * HID), lambda i: (0, 0)),
        ],
        out_specs=pl.BlockSpec((VTILE, 2 * HID), lambda i: (i, 0)),
        out_shape=jax.ShapeDtypeStruct((VOCAB // 2, 2 * HID), jnp.float32),
    )(emb2, w2ih, b2ih)

    idx = jnp.transpose(x).reshape(TOK // SUB, SUB)

    table = proj2.reshape(VOCAB, HID)
    w2hh = _dup_diag(W_hh.T)
    b2hh = jnp.concatenate([b_hh, b_hh]).reshape(1, 2 * HID)
    zf = jnp.zeros((HID,), jnp.float32)
    flo = jnp.concatenate([fc_W[0], zf]).reshape(1, 2 * HID)
    fhi = jnp.concatenate([zf, fc_W[0]]).reshape(1, 2 * HID)
    fcb2 = jnp.broadcast_to(fc_b.reshape(1, 1), (1, 2))

    toks = B * SEG
    irows = toks // SUB
    gs = [
        _sc_gather_fn(toks)(table, lax.slice_in_dim(idx, s * irows, (s + 1) * irows))
        for s in range(NSEG)
    ]

    h = jnp.zeros((B // 2, 2 * HID), jnp.float32)
    out = None
    for s in range(NSEG):
        g2 = gs[s].reshape(toks // 2, 2 * HID)
        h, out = pl.pallas_call(
            _scan_seg_kernel,
            grid=(SEG,),
            in_specs=[
                pl.BlockSpec((B // 2, 2 * HID), lambda t: (t, 0)),
                pl.BlockSpec((2 * HID, 2 * HID), lambda t: (0, 0)),
                pl.BlockSpec((1, 2 * HID), lambda t: (0, 0)),
                pl.BlockSpec((1, 2 * HID), lambda t: (0, 0)),
                pl.BlockSpec((1, 2 * HID), lambda t: (0, 0)),
                pl.BlockSpec((1, 2), lambda t: (0, 0)),
                pl.BlockSpec((B // 2, 2 * HID), lambda t: (0, 0)),
            ],
            out_specs=[
                pl.BlockSpec((B // 2, 2 * HID), lambda t: (0, 0)),
                pl.BlockSpec((B // 2, 2), lambda t: (0, 0)),
            ],
            out_shape=[
                jax.ShapeDtypeStruct((B // 2, 2 * HID), jnp.float32),
                jax.ShapeDtypeStruct((B // 2, 2), jnp.float32),
            ],
            scratch_shapes=[pltpu.VMEM((B // 2, 2 * HID), jnp.float32)],
        )(g2, w2hh, b2hh, flo, fhi, fcb2, h)
    return out.reshape(B)


# speedup vs baseline: 2.0179x; 1.0014x over previous
import functools

import jax
import jax.numpy as jnp
from jax import lax
from jax.experimental import pallas as pl
from jax.experimental.pallas import tpu as pltpu
from jax.experimental.pallas import tpu_sc as plsc

VOCAB = 100000
EMB = 64
HID = 64
B = 4096
S = 200

NC = 2
NS = 16
NW = NC * NS
TOK = B * S
TPW = TOK // NW
SUB = 128
KSUB = 4
CHUNK = SUB * KSUB

VTILE = 2000

SEGS = (8, 16, 32, 64, 80)


def _project_kernel(emb_ref, wt_ref, b_ref, out_ref):
    out_ref[...] = jnp.dot(
        emb_ref[...], wt_ref[...], preferred_element_type=jnp.float32
    ) + b_ref[...]


@functools.cache
def _sc_gather_fn(ntok):
    tpw = ntok // NW
    nchunk = tpw // CHUNK
    mesh = plsc.VectorSubcoreMesh(
        core_axis_name="c", subcore_axis_name="s", num_cores=NC, num_subcores=NS
    )

    @functools.partial(
        pl.kernel,
        out_type=jax.ShapeDtypeStruct((ntok, HID), jnp.float32),
        mesh=mesh,
        scratch_types=[
            pltpu.VMEM((2 * KSUB, SUB), jnp.int32),
            pltpu.VMEM((2 * CHUNK, HID), jnp.float32),
            pltpu.SemaphoreType.DMA((2,)),
            pltpu.SemaphoreType.DMA((2,)),
            pltpu.SemaphoreType.DMA((2,)),
        ],
        compiler_params=pltpu.CompilerParams(use_tc_tiling_on_sc=False),
    )
    def _sc_gather(table_hbm, idx_hbm, out_hbm, idx_v, rows_v, isem, gsem, osem):
        wid = lax.axis_index("s") * NC + lax.axis_index("c")
        row_base = wid * (tpw // SUB)
        NCHUNK = nchunk

        def fire_idx(i, b):
            pltpu.async_copy(
                idx_hbm.at[pl.ds(row_base + i * KSUB, KSUB)],
                idx_v.at[pl.ds(b * KSUB, KSUB)],
                isem.at[b],
            )

        def wait_idx(b):
            pltpu.make_async_copy(
                idx_hbm.at[pl.ds(row_base, KSUB)],
                idx_v.at[pl.ds(b * KSUB, KSUB)],
                isem.at[b],
            ).wait()

        def fire_gathers(b):
            for j in range(KSUB):
                pltpu.async_copy(
                    table_hbm.at[idx_v.at[b * KSUB + j]],
                    rows_v.at[pl.ds((b * KSUB + j) * SUB, SUB)],
                    gsem.at[b],
                )

        def wait_gathers(b):
            pltpu.make_async_copy(
                out_hbm.at[pl.ds(0, CHUNK)],
                rows_v.at[pl.ds(b * CHUNK, CHUNK)],
                gsem.at[b],
            ).wait()

        def fire_scatter(i, b):
            pltpu.async_copy(
                rows_v.at[pl.ds(b * CHUNK, CHUNK)],
                out_hbm.at[pl.ds((row_base + i * KSUB) * SUB, CHUNK)],
                osem.at[b],
            )

        def wait_scatter(b):
            pltpu.make_async_copy(
                rows_v.at[pl.ds(b * CHUNK, CHUNK)],
                out_hbm.at[pl.ds(0, CHUNK)],
                osem.at[b],
            ).wait()

        fire_idx(0, 0)
        fire_idx(1, 1)
        wait_idx(0)
        fire_gathers(0)

        @pl.loop(0, NCHUNK)
        def _chunk(i):
            b = lax.rem(i, 2)
            nxt = 1 - b

            @pl.when(i + 1 < NCHUNK)
            def _():
                @pl.when(i >= 1)
                def _():
                    wait_scatter(nxt)

                wait_idx(nxt)
                fire_gathers(nxt)

            wait_gathers(b)

            @pl.when(i + 2 < NCHUNK)
            def _():
                fire_idx(i + 2, b)

            fire_scatter(i, b)

        wait_scatter(0)
        wait_scatter(1)

    return _sc_gather


def _scan_seg_kernel(
    g_ref, w2_ref, b2_ref, flo_ref, fhi_ref, fcb_ref, hin_ref,
    hout_ref, out_ref, h_ref
):
    t = pl.program_id(0)
    nt = pl.num_programs(0)

    @pl.when(t == 0)
    def _():
        h_ref[...] = hin_ref[...]

    h_new = jnp.tanh(
        g_ref[...]
        + jnp.dot(h_ref[...], w2_ref[...], preferred_element_type=jnp.float32)
        + b2_ref[...]
    )
    h_ref[...] = h_new

    @pl.when(t == nt - 1)
    def _():
        hout_ref[...] = h_new
        lo = jnp.sum(h_new * flo_ref[...], axis=1, keepdims=True)
        hi = jnp.sum(h_new * fhi_ref[...], axis=1, keepdims=True)
        logits = jnp.concatenate([lo, hi], axis=1) + fcb_ref[...]
        out_ref[...] = jax.nn.sigmoid(logits)


def _dup_diag(wt):
    z = jnp.zeros_like(wt)
    return jnp.concatenate(
        [jnp.concatenate([wt, z], axis=1), jnp.concatenate([z, wt], axis=1)],
        axis=0,
    )


def kernel(x, emb, W_ih, W_hh, b_ih, b_hh, fc_W, fc_b):
    emb2 = emb.reshape(VOCAB // 2, 2 * EMB)
    w2ih = _dup_diag(W_ih.T)
    b2ih = jnp.concatenate([b_ih, b_ih]).reshape(1, 2 * HID)

    proj2 = pl.pallas_call(
        _project_kernel,
        grid=(VOCAB // 2 // VTILE,),
        in_specs=[
            pl.BlockSpec((VTILE, 2 * EMB), lambda i: (i, 0)),
            pl.BlockSpec((2 * EMB, 2 * HID), lambda i: (0, 0)),
            pl.BlockSpec((1, 2 * HID), lambda i: (0, 0)),
        ],
        out_specs=pl.BlockSpec((VTILE, 2 * HID), lambda i: (i, 0)),
        out_shape=jax.ShapeDtypeStruct((VOCAB // 2, 2 * HID), jnp.float32),
    )(emb2, w2ih, b2ih)

    idx = jnp.transpose(x).reshape(TOK // SUB, SUB)

    table = proj2.reshape(VOCAB, HID)
    w2hh = _dup_diag(W_hh.T)
    b2hh = jnp.concatenate([b_hh, b_hh]).reshape(1, 2 * HID)
    zf = jnp.zeros((HID,), jnp.float32)
    flo = jnp.concatenate([fc_W[0], zf]).reshape(1, 2 * HID)
    fhi = jnp.concatenate([zf, fc_W[0]]).reshape(1, 2 * HID)
    fcb2 = jnp.broadcast_to(fc_b.reshape(1, 1), (1, 2))

    bounds = [0]
    for seg in SEGS:
        bounds.append(bounds[-1] + seg)
    gs = []
    for s, seg in enumerate(SEGS):
        irow0 = bounds[s] * B // SUB
        irow1 = bounds[s + 1] * B // SUB
        gs.append(
            _sc_gather_fn(seg * B)(table, lax.slice_in_dim(idx, irow0, irow1))
        )

    h = jnp.zeros((B // 2, 2 * HID), jnp.float32)
    out = None
    for s, seg in enumerate(SEGS):
        toks = seg * B
        g2 = gs[s].reshape(toks // 2, 2 * HID)
        h, out = pl.pallas_call(
            _scan_seg_kernel,
            grid=(seg,),
            in_specs=[
                pl.BlockSpec((B // 2, 2 * HID), lambda t: (t, 0)),
                pl.BlockSpec((2 * HID, 2 * HID), lambda t: (0, 0)),
                pl.BlockSpec((1, 2 * HID), lambda t: (0, 0)),
                pl.BlockSpec((1, 2 * HID), lambda t: (0, 0)),
                pl.BlockSpec((1, 2 * HID), lambda t: (0, 0)),
                pl.BlockSpec((1, 2), lambda t: (0, 0)),
                pl.BlockSpec((B // 2, 2 * HID), lambda t: (0, 0)),
            ],
            out_specs=[
                pl.BlockSpec((B // 2, 2 * HID), lambda t: (0, 0)),
                pl.BlockSpec((B // 2, 2), lambda t: (0, 0)),
            ],
            out_shape=[
                jax.ShapeDtypeStruct((B // 2, 2 * HID), jnp.float32),
                jax.ShapeDtypeStruct((B // 2, 2), jnp.float32),
            ],
            scratch_shapes=[pltpu.VMEM((B // 2, 2 * HID), jnp.float32)],
        )(g2, w2hh, b2hh, flo, fhi, fcb2, h)
    return out.reshape(B)


# speedup vs baseline: 2.0870x; 1.0343x over previous
import functools

import jax
import jax.numpy as jnp
from jax import lax
from jax.experimental import pallas as pl
from jax.experimental.pallas import tpu as pltpu
from jax.experimental.pallas import tpu_sc as plsc

VOCAB = 100000
EMB = 64
HID = 64
B = 4096
S = 200

NC = 2
NS = 16
NW = NC * NS
TOK = B * S
TPW = TOK // NW
SUB = 128
KSUB = 4
CHUNK = SUB * KSUB

VTILE = 2000

SEGS = (8, 14, 22, 34, 52, 70)


def _project_kernel(emb_ref, wt_ref, b_ref, out_ref):
    out_ref[...] = jnp.dot(
        emb_ref[...], wt_ref[...], preferred_element_type=jnp.float32
    ) + b_ref[...]


@functools.cache
def _sc_gather_fn(ntok):
    tpw = ntok // NW
    nchunk = tpw // CHUNK
    mesh = plsc.VectorSubcoreMesh(
        core_axis_name="c", subcore_axis_name="s", num_cores=NC, num_subcores=NS
    )

    @functools.partial(
        pl.kernel,
        out_type=jax.ShapeDtypeStruct((ntok, HID), jnp.float32),
        mesh=mesh,
        scratch_types=[
            pltpu.VMEM((2 * KSUB, SUB), jnp.int32),
            pltpu.VMEM((2 * CHUNK, HID), jnp.float32),
            pltpu.SemaphoreType.DMA((2,)),
            pltpu.SemaphoreType.DMA((2,)),
            pltpu.SemaphoreType.DMA((2,)),
        ],
        compiler_params=pltpu.CompilerParams(use_tc_tiling_on_sc=False),
    )
    def _sc_gather(table_hbm, idx_hbm, out_hbm, idx_v, rows_v, isem, gsem, osem):
        wid = lax.axis_index("s") * NC + lax.axis_index("c")
        row_base = wid * (tpw // SUB)
        NCHUNK = nchunk

        def fire_idx(i, b):
            pltpu.async_copy(
                idx_hbm.at[pl.ds(row_base + i * KSUB, KSUB)],
                idx_v.at[pl.ds(b * KSUB, KSUB)],
                isem.at[b],
            )

        def wait_idx(b):
            pltpu.make_async_copy(
                idx_hbm.at[pl.ds(row_base, KSUB)],
                idx_v.at[pl.ds(b * KSUB, KSUB)],
                isem.at[b],
            ).wait()

        def fire_gathers(b):
            for j in range(KSUB):
                pltpu.async_copy(
                    table_hbm.at[idx_v.at[b * KSUB + j]],
                    rows_v.at[pl.ds((b * KSUB + j) * SUB, SUB)],
                    gsem.at[b],
                )

        def wait_gathers(b):
            pltpu.make_async_copy(
                out_hbm.at[pl.ds(0, CHUNK)],
                rows_v.at[pl.ds(b * CHUNK, CHUNK)],
                gsem.at[b],
            ).wait()

        def fire_scatter(i, b):
            pltpu.async_copy(
                rows_v.at[pl.ds(b * CHUNK, CHUNK)],
                out_hbm.at[pl.ds((row_base + i * KSUB) * SUB, CHUNK)],
                osem.at[b],
            )

        def wait_scatter(b):
            pltpu.make_async_copy(
                rows_v.at[pl.ds(b * CHUNK, CHUNK)],
                out_hbm.at[pl.ds(0, CHUNK)],
                osem.at[b],
            ).wait()

        fire_idx(0, 0)
        fire_idx(1, 1)
        wait_idx(0)
        fire_gathers(0)

        @pl.loop(0, NCHUNK)
        def _chunk(i):
            b = lax.rem(i, 2)
            nxt = 1 - b

            @pl.when(i + 1 < NCHUNK)
            def _():
                @pl.when(i >= 1)
                def _():
                    wait_scatter(nxt)

                wait_idx(nxt)
                fire_gathers(nxt)

            wait_gathers(b)

            @pl.when(i + 2 < NCHUNK)
            def _():
                fire_idx(i + 2, b)

            fire_scatter(i, b)

        wait_scatter(0)
        wait_scatter(1)

    return _sc_gather


def _scan_seg_kernel(
    g_ref, w2_ref, b2_ref, flo_ref, fhi_ref, fcb_ref, hin_ref,
    hout_ref, out_ref, h_ref
):
    t = pl.program_id(0)
    nt = pl.num_programs(0)

    @pl.when(t == 0)
    def _():
        h_ref[...] = hin_ref[...]

    h_new = jnp.tanh(
        g_ref[...]
        + jnp.dot(h_ref[...], w2_ref[...], preferred_element_type=jnp.float32)
        + b2_ref[...]
    )
    h_ref[...] = h_new

    @pl.when(t == nt - 1)
    def _():
        hout_ref[...] = h_new
        lo = jnp.sum(h_new * flo_ref[...], axis=1, keepdims=True)
        hi = jnp.sum(h_new * fhi_ref[...], axis=1, keepdims=True)
        logits = jnp.concatenate([lo, hi], axis=1) + fcb_ref[...]
        out_ref[...] = jax.nn.sigmoid(logits)


def _dup_diag(wt):
    z = jnp.zeros_like(wt)
    return jnp.concatenate(
        [jnp.concatenate([wt, z], axis=1), jnp.concatenate([z, wt], axis=1)],
        axis=0,
    )


def kernel(x, emb, W_ih, W_hh, b_ih, b_hh, fc_W, fc_b):
    emb2 = emb.reshape(VOCAB // 2, 2 * EMB)
    w2ih = _dup_diag(W_ih.T)
    b2ih = jnp.concatenate([b_ih, b_ih]).reshape(1, 2 * HID)

    proj2 = pl.pallas_call(
        _project_kernel,
        grid=(VOCAB // 2 // VTILE,),
        in_specs=[
            pl.BlockSpec((VTILE, 2 * EMB), lambda i: (i, 0)),
            pl.BlockSpec((2 * EMB, 2 * HID), lambda i: (0, 0)),
            pl.BlockSpec((1, 2 * HID), lambda i: (0, 0)),
        ],
        out_specs=pl.BlockSpec((VTILE, 2 * HID), lambda i: (i, 0)),
        out_shape=jax.ShapeDtypeStruct((VOCAB // 2, 2 * HID), jnp.float32),
    )(emb2, w2ih, b2ih)

    table = proj2.reshape(VOCAB, HID)
    w2hh = _dup_diag(W_hh.T)
    b2hh = jnp.concatenate([b_hh, b_hh]).reshape(1, 2 * HID)
    zf = jnp.zeros((HID,), jnp.float32)
    flo = jnp.concatenate([fc_W[0], zf]).reshape(1, 2 * HID)
    fhi = jnp.concatenate([zf, fc_W[0]]).reshape(1, 2 * HID)
    fcb2 = jnp.broadcast_to(fc_b.reshape(1, 1), (1, 2))

    bounds = [0]
    for seg in SEGS:
        bounds.append(bounds[-1] + seg)
    gs = []
    for s, seg in enumerate(SEGS):
        idx_s = jnp.transpose(
            lax.slice_in_dim(x, bounds[s], bounds[s + 1], axis=1)
        ).reshape(seg * B // SUB, SUB)
        gs.append(_sc_gather_fn(seg * B)(table, idx_s))

    h = jnp.zeros((B // 2, 2 * HID), jnp.float32)
    out = None
    for s, seg in enumerate(SEGS):
        toks = seg * B
        g2 = gs[s].reshape(toks // 2, 2 * HID)
        h, out = pl.pallas_call(
            _scan_seg_kernel,
            grid=(seg,),
            in_specs=[
                pl.BlockSpec((B // 2, 2 * HID), lambda t: (t, 0)),
                pl.BlockSpec((2 * HID, 2 * HID), lambda t: (0, 0)),
                pl.BlockSpec((1, 2 * HID), lambda t: (0, 0)),
                pl.BlockSpec((1, 2 * HID), lambda t: (0, 0)),
                pl.BlockSpec((1, 2 * HID), lambda t: (0, 0)),
                pl.BlockSpec((1, 2), lambda t: (0, 0)),
                pl.BlockSpec((B // 2, 2 * HID), lambda t: (0, 0)),
            ],
            out_specs=[
                pl.BlockSpec((B // 2, 2 * HID), lambda t: (0, 0)),
                pl.BlockSpec((B // 2, 2), lambda t: (0, 0)),
            ],
            out_shape=[
                jax.ShapeDtypeStruct((B // 2, 2 * HID), jnp.float32),
                jax.ShapeDtypeStruct((B // 2, 2), jnp.float32),
            ],
            scratch_shapes=[pltpu.VMEM((B // 2, 2 * HID), jnp.float32)],
        )(g2, w2hh, b2hh, flo, fhi, fcb2, h)
    return out.reshape(B)
